# baseline (speedup 1.0000x reference)
import functools

import jax
import jax.numpy as jnp
from jax import lax
from jax.experimental import pallas as pl
from jax.experimental.pallas import tpu as pltpu
from jax.experimental.pallas import tpu_sc as plsc

N_SPECIES = 4
LANES = 16
NTILES = 32
BLK = 512


def _mesh():
    return plsc.VectorSubcoreMesh(core_axis_name="c", subcore_axis_name="s")


def _wid():
    return lax.axis_index("s") * 2 + lax.axis_index("c")


def _lane_iota():
    return lax.iota(jnp.int32, LANES)


def _take(x, idx):
    dnums = lax.GatherDimensionNumbers(
        offset_dims=(), collapsed_slice_dims=(0,), start_index_map=(0,))
    return lax.gather(x, idx[:, None], dnums, slice_sizes=(1,),
                      mode=lax.GatherScatterMode.PROMISE_IN_BOUNDS)


def _incl_prefix(x):
    iot = _lane_iota()
    for sh in (1, 2, 4, 8):
        shifted = _take(x, jnp.maximum(iot - sh, 0))
        x = x + jnp.where(iot >= sh, shifted, 0.0)
    return x


def _lanesum_splat(x):
    return _take(_incl_prefix(x), jnp.full((LANES,), LANES - 1, jnp.int32))


def _make_counts_kernel(n):
    per = n // NTILES

    @functools.partial(
        pl.kernel,
        out_type=jax.ShapeDtypeStruct((NTILES, N_SPECIES, LANES), jnp.float32),
        mesh=_mesh(),
        scratch_types=[
            pltpu.VMEM((per,), jnp.int32),
            pltpu.VMEM((N_SPECIES, LANES), jnp.float32),
        ],
    )
    def counts_kernel(spec_hbm, counts_hbm, spec_v, cnt_v):
        wid = _wid()
        base = wid * per
        pltpu.sync_copy(spec_hbm.at[pl.ds(base, per)], spec_v)

        def body(i, carry):
            v = spec_v[pl.ds(i * LANES, LANES)]
            return tuple(carry[s] + jnp.where(v == s, 1.0, 0.0)
                         for s in range(N_SPECIES))

        zeros = jnp.zeros((LANES,), jnp.float32)
        cnts = lax.fori_loop(0, per // LANES, body, (zeros,) * N_SPECIES)
        for s in range(N_SPECIES):
            cnt_v[s, :] = cnts[s]
        pltpu.sync_copy(cnt_v, counts_hbm.at[wid])

    return counts_kernel


def _make_scatter_kernel(n, aev_dim):
    per = n // NTILES
    sup = 128
    nsup = per // sup
    nchunk = sup // LANES

    @functools.partial(
        pl.kernel,
        out_type=(
            jax.ShapeDtypeStruct((n, aev_dim), jnp.float32),
            jax.ShapeDtypeStruct((n,), jnp.int32),
            jax.ShapeDtypeStruct((LANES,), jnp.int32),
        ),
        mesh=_mesh(),
        scratch_types=[
            pltpu.VMEM((per,), jnp.int32),
            pltpu.VMEM((per,), jnp.int32),
            pltpu.VMEM((NTILES, N_SPECIES, LANES), jnp.float32),
            pltpu.VMEM((2, sup, aev_dim), jnp.float32),
            pltpu.VMEM((LANES,), jnp.int32),
            pltpu.SemaphoreType.DMA,
            pltpu.SemaphoreType.DMA,
            pltpu.SemaphoreType.DMA,
        ],
    )
    def scatter_kernel(spec_hbm, aev_hbm, counts_hbm,
                       sorted_hbm, dest_hbm, offs_hbm,
                       spec_v, dest_v, cnt_v, rows_v, offs_v,
                       sem_a, sem_b, sem_s):
        wid = _wid()
        base = wid * per
        pltpu.sync_copy(spec_hbm.at[pl.ds(base, per)], spec_v)
        pltpu.sync_copy(counts_hbm, cnt_v)

        zero = jnp.zeros((LANES,), jnp.float32)
        tot = [zero] * N_SPECIES
        before = [zero] * N_SPECIES
        for w in range(NTILES):
            sel = jnp.where(jnp.int32(w) < wid, 1.0, 0.0)
            for s in range(N_SPECIES):
                v = cnt_v[w, s, :]
                tot[s] = tot[s] + v
                before[s] = before[s] + v * sel
        tot_s = [_lanesum_splat(tot[s]) for s in range(N_SPECIES)]
        start_s = []
        acc = jnp.zeros((LANES,), jnp.float32)
        for s in range(N_SPECIES):
            start_s.append(acc)
            acc = acc + tot_s[s]
        base_s = tuple(start_s[s] + _lanesum_splat(before[s])
                       for s in range(N_SPECIES))

        iot = lax.iota(jnp.int32, LANES)
        offs = jnp.where(iot == N_SPECIES, acc, 0.0)
        for s in range(N_SPECIES):
            offs = offs + jnp.where(iot == s, start_s[s], 0.0)
        offs_v[...] = offs.astype(jnp.int32)

        @pl.when(wid == 0)
        def _():
            pltpu.sync_copy(offs_v, offs_hbm)

        def load(g, buf):
            return pltpu.make_async_copy(
                aev_hbm.at[pl.ds(base + g * sup, sup)], rows_v.at[buf],
                sem_a if buf == 0 else sem_b)

        load(0, 0).start()

        def process(g, buf, bases):
            handles = []
            for j in range(nchunk):
                c = g * nchunk + j
                v = spec_v[pl.ds(c * LANES, LANES)]
                dest = jnp.zeros((LANES,), jnp.float32)
                new_bases = []
                for s in range(N_SPECIES):
                    mi = jnp.where(v == s, 1.0, 0.0)
                    ipfx = _incl_prefix(mi)
                    dest = dest + mi * (bases[s] + ipfx - mi)
                    new_bases.append(
                        bases[s] + _take(ipfx, jnp.full((LANES,), LANES - 1,
                                                        jnp.int32)))
                bases = tuple(new_bases)
                dest_i = dest.astype(jnp.int32)
                dest_v[pl.ds(c * LANES, LANES)] = dest_i
                handles.append(pltpu.async_copy(
                    rows_v.at[buf].at[pl.ds(j * LANES, LANES)],
                    sorted_hbm.at[dest_i], sem_s))
            for h in handles:
                h.wait()
            return bases

        def pair(p, bases):
            g0 = p * 2

            @pl.when(g0 + 1 < nsup)
            def _():
                load(g0 + 1, 1).start()

            load(g0, 0).wait()
            bases = process(g0, 0, bases)

            @pl.when(g0 + 2 < nsup)
            def _():
                load(g0 + 2, 0).start()

            load(g0 + 1, 1).wait()
            bases = process(g0 + 1, 1, bases)
            return bases

        lax.fori_loop(0, nsup // 2, pair, base_s)
        pltpu.sync_copy(dest_v, dest_hbm.at[pl.ds(base, per)])

    return scatter_kernel


_LOG2E10 = 14.426950408889634


def _celu_shift(x):
    return jnp.maximum(x, 0.0) + 0.1 * jnp.exp2(jnp.minimum(x, 0.0) * _LOG2E10)


def _grouped_mlp_kernel(offs_ref, x_ref, *refs):
    out_ref = refs[-1]
    wrefs = refs[:-1]
    x = x_ref[...]
    row0 = pl.program_id(0) * BLK
    rows = row0 + lax.broadcasted_iota(jnp.int32, (BLK, 1), 0)
    out_ref[...] = jnp.zeros_like(out_ref)
    for s in range(N_SPECIES):
        lo = offs_ref[s]
        hi = offs_ref[s + 1]

        @pl.when(jnp.logical_and(hi > row0, lo < row0 + BLK))
        def _(s=s, lo=lo, hi=hi):
            W0, b0, W1, b1, W2, b2 = (r[...] for r in wrefs[6 * s:6 * s + 6])
            h = _celu_shift(lax.dot_general(x, W0, (((1,), (1,)), ((), ())),
                                            preferred_element_type=jnp.float32) + b0)
            h = _celu_shift(lax.dot_general(h, W1, (((1,), (1,)), ((), ())),
                                            preferred_element_type=jnp.float32) + b1)
            h = _celu_shift(lax.dot_general(h, W2, (((1,), (1,)), ((), ())),
                                            preferred_element_type=jnp.float32) + b2)
            m = jnp.logical_and(rows >= lo, rows < hi).astype(jnp.float32)
            out_ref[...] += h * m

    out_ref[...] -= 0.1


def _grouped_mlp(sorted_aevs, offs, weights, n_feats):
    n, aev_dim = sorted_aevs.shape
    nblk = n // BLK

    def w_spec(w):
        return pl.BlockSpec(w.shape, lambda i: (0,) * w.ndim)

    return pl.pallas_call(
        _grouped_mlp_kernel,
        grid=(nblk,),
        in_specs=[
            pl.BlockSpec(memory_space=pltpu.SMEM),
            pl.BlockSpec((BLK, aev_dim), lambda i: (i, 0)),
        ] + [w_spec(w) for w in weights],
        out_specs=pl.BlockSpec((BLK, n_feats), lambda i: (i, 0)),
        out_shape=jax.ShapeDtypeStruct((n, n_feats), jnp.float32),
    )(offs, sorted_aevs, *weights)


def _make_gather_kernel(n, n_feats, n_pad):
    per = n // NTILES
    sup = 128
    nsup = per // sup

    @functools.partial(
        pl.kernel,
        out_type=jax.ShapeDtypeStruct((n, n_pad), jnp.float32),
        mesh=_mesh(),
        scratch_types=[
            pltpu.VMEM((per,), jnp.int32),
            pltpu.VMEM((2, sup, n_pad), jnp.float32),
            pltpu.SemaphoreType.DMA,
            pltpu.SemaphoreType.DMA,
            pltpu.SemaphoreType.DMA,
        ],
    )
    def gather_kernel(feats_hbm, dest_hbm, out_hbm, dest_v, rows_v,
                      sem_a, sem_b, sem_o):
        wid = _wid()
        base = wid * per
        pltpu.sync_copy(dest_hbm.at[pl.ds(base, per)], dest_v)

        def gath(g, buf):
            return pltpu.make_async_copy(
                feats_hbm.at[dest_v.at[pl.ds(g * sup, sup)]], rows_v.at[buf],
                sem_a if buf == 0 else sem_b)

        def store(g, buf):
            return pltpu.make_async_copy(
                rows_v.at[buf], out_hbm.at[pl.ds(base + g * sup, sup)], sem_o)

        gath(0, 0).start()

        def pair(p, _):
            g0 = p * 2

            @pl.when(g0 + 1 < nsup)
            def _():
                gath(g0 + 1, 1).start()

            gath(g0, 0).wait()
            store(g0, 0).start()
            store(g0, 0).wait()

            @pl.when(g0 + 2 < nsup)
            def _():
                gath(g0 + 2, 0).start()

            gath(g0 + 1, 1).wait()
            store(g0 + 1, 1).start()
            store(g0 + 1, 1).wait()
            return 0

        lax.fori_loop(0, nsup // 2, pair, 0)

    return gather_kernel


def kernel(species, aevs, W0_s0, b0_s0, W1_s0, b1_s0, W2_s0, b2_s0,
           W0_s1, b0_s1, W1_s1, b1_s1, W2_s1, b2_s1,
           W0_s2, b0_s2, W1_s2, b1_s2, W2_s2, b2_s2,
           W0_s3, b0_s3, W1_s3, b1_s3, W2_s3, b2_s3):
    b, a = species.shape
    n = b * a
    aev_dim = aevs.shape[-1]
    n_feats = W2_s0.shape[0]

    spec_flat = species.reshape(n)
    flat = aevs.reshape(n, aev_dim)
    weights = (W0_s0, b0_s0, W1_s0, b1_s0, W2_s0, b2_s0,
               W0_s1, b0_s1, W1_s1, b1_s1, W2_s1, b2_s1,
               W0_s2, b0_s2, W1_s2, b1_s2, W2_s2, b2_s2,
               W0_s3, b0_s3, W1_s3, b1_s3, W2_s3, b2_s3)

    n_pad = 128
    pw = n_pad - n_feats
    weights = list(weights)
    for i in range(4, 24, 6):
        weights[i] = jnp.pad(weights[i], ((0, pw), (0, 0)))
        weights[i + 1] = jnp.pad(weights[i + 1], ((0, pw),))
    for i in range(0, 24, 6):
        for wj, bj in ((i + 2, i + 3), (i + 4, i + 5)):
            weights[bj] = weights[bj] - 0.1 * jnp.sum(weights[wj], axis=1)
    weights = tuple(weights)

    counts = _make_counts_kernel(n)(spec_flat)
    sorted_aevs, dest, offs = _make_scatter_kernel(n, aev_dim)(
        spec_flat, flat, counts)
    sorted_feats = _grouped_mlp(sorted_aevs, offs, weights, n_pad)
    final = _make_gather_kernel(n, n_feats, n_pad)(sorted_feats, dest)

    return species, final[:, :n_feats].reshape(b, a, n_feats)


# speedup vs baseline: 1.3385x; 1.3385x over previous
import functools

import jax
import jax.numpy as jnp
from jax import lax
from jax.experimental import pallas as pl
from jax.experimental.pallas import tpu as pltpu
from jax.experimental.pallas import tpu_sc as plsc

N_SPECIES = 4
LANES = 16
NTILES = 32
BLK = 8192


def _mesh():
    return plsc.VectorSubcoreMesh(core_axis_name="c", subcore_axis_name="s")


def _wid():
    return lax.axis_index("s") * 2 + lax.axis_index("c")


def _lane_iota():
    return lax.iota(jnp.int32, LANES)


def _take(x, idx):
    dnums = lax.GatherDimensionNumbers(
        offset_dims=(), collapsed_slice_dims=(0,), start_index_map=(0,))
    return lax.gather(x, idx[:, None], dnums, slice_sizes=(1,),
                      mode=lax.GatherScatterMode.PROMISE_IN_BOUNDS)


def _incl_prefix(x):
    iot = _lane_iota()
    for sh in (1, 2, 4, 8):
        shifted = _take(x, jnp.maximum(iot - sh, 0))
        x = x + jnp.where(iot >= sh, shifted, 0.0)
    return x


def _lanesum_splat(x):
    return _take(_incl_prefix(x), jnp.full((LANES,), LANES - 1, jnp.int32))


def _make_counts_kernel(n):
    per = n // NTILES

    @functools.partial(
        pl.kernel,
        out_type=jax.ShapeDtypeStruct((NTILES, N_SPECIES, LANES), jnp.float32),
        mesh=_mesh(),
        scratch_types=[
            pltpu.VMEM((per,), jnp.int32),
            pltpu.VMEM((N_SPECIES, LANES), jnp.float32),
        ],
    )
    def counts_kernel(spec_hbm, counts_hbm, spec_v, cnt_v):
        wid = _wid()
        base = wid * per
        pltpu.sync_copy(spec_hbm.at[pl.ds(base, per)], spec_v)

        def body(i, carry):
            v = spec_v[pl.ds(i * LANES, LANES)]
            return tuple(carry[s] + jnp.where(v == s, 1.0, 0.0)
                         for s in range(N_SPECIES))

        zeros = jnp.zeros((LANES,), jnp.float32)
        cnts = lax.fori_loop(0, per // LANES, body, (zeros,) * N_SPECIES)
        for s in range(N_SPECIES):
            cnt_v[s, :] = cnts[s]
        pltpu.sync_copy(cnt_v, counts_hbm.at[wid])

    return counts_kernel


def _make_scatter_kernel(n, aev_dim, npad, tbl_len):
    per = n // NTILES
    sup = 128
    nsup = per // sup
    nchunk = sup // LANES

    @functools.partial(
        pl.kernel,
        out_type=(
            jax.ShapeDtypeStruct((npad, aev_dim), jnp.float32),
            jax.ShapeDtypeStruct((n,), jnp.int32),
            jax.ShapeDtypeStruct((tbl_len,), jnp.int32),
        ),
        mesh=_mesh(),
        scratch_types=[
            pltpu.VMEM((per,), jnp.int32),
            pltpu.VMEM((per,), jnp.int32),
            pltpu.VMEM((NTILES, N_SPECIES, LANES), jnp.float32),
            pltpu.VMEM((2, sup, aev_dim), jnp.float32),
            pltpu.VMEM((tbl_len,), jnp.int32),
            pltpu.SemaphoreType.DMA,
            pltpu.SemaphoreType.DMA,
            pltpu.SemaphoreType.DMA,
        ],
    )
    def scatter_kernel(spec_hbm, aev_hbm, counts_hbm,
                       sorted_hbm, dest_hbm, tbl_hbm,
                       spec_v, dest_v, cnt_v, rows_v, tbl_v,
                       sem_a, sem_b, sem_s):
        wid = _wid()
        base = wid * per
        pltpu.sync_copy(spec_hbm.at[pl.ds(base, per)], spec_v)
        pltpu.sync_copy(counts_hbm, cnt_v)

        zero = jnp.zeros((LANES,), jnp.float32)
        tot = [zero] * N_SPECIES
        before = [zero] * N_SPECIES
        for w in range(NTILES):
            sel = jnp.where(jnp.int32(w) < wid, 1.0, 0.0)
            for s in range(N_SPECIES):
                v = cnt_v[w, s, :]
                tot[s] = tot[s] + v
                before[s] = before[s] + v * sel
        tot_s = [_lanesum_splat(tot[s]) for s in range(N_SPECIES)]
        blk_f = jnp.float32(BLK)
        start_s = []
        acc = jnp.zeros((LANES,), jnp.float32)
        for s in range(N_SPECIES):
            start_s.append(acc)
            r = lax.rem(tot_s[s], blk_f)
            acc = acc + tot_s[s] + jnp.where(r > 0, blk_f - r, 0.0)
        base_s = tuple(start_s[s] + _lanesum_splat(before[s])
                       for s in range(N_SPECIES))

        iot = lax.iota(jnp.int32, LANES)
        for t in range(tbl_len // LANES):
            brow = (iot + t * LANES).astype(jnp.float32) * blk_f
            sp = jnp.zeros((LANES,), jnp.float32)
            for s in range(1, N_SPECIES):
                sp = sp + jnp.where(brow >= start_s[s], 1.0, 0.0)
            tbl_v[pl.ds(t * LANES, LANES)] = sp.astype(jnp.int32)

        @pl.when(wid == 0)
        def _():
            pltpu.sync_copy(tbl_v, tbl_hbm)

        def load(g, buf):
            return pltpu.make_async_copy(
                aev_hbm.at[pl.ds(base + g * sup, sup)], rows_v.at[buf],
                sem_a if buf == 0 else sem_b)

        load(0, 0).start()

        def process(g, buf, bases):
            handles = []
            for j in range(nchunk):
                c = g * nchunk + j
                v = spec_v[pl.ds(c * LANES, LANES)]
                dest = jnp.zeros((LANES,), jnp.float32)
                new_bases = []
                for s in range(N_SPECIES):
                    mi = jnp.where(v == s, 1.0, 0.0)
                    ipfx = _incl_prefix(mi)
                    dest = dest + mi * (bases[s] + ipfx - mi)
                    new_bases.append(
                        bases[s] + _take(ipfx, jnp.full((LANES,), LANES - 1,
                                                        jnp.int32)))
                bases = tuple(new_bases)
                dest_i = dest.astype(jnp.int32)
                dest_v[pl.ds(c * LANES, LANES)] = dest_i
                handles.append(pltpu.async_copy(
                    rows_v.at[buf].at[pl.ds(j * LANES, LANES)],
                    sorted_hbm.at[dest_i], sem_s))
            for h in handles:
                h.wait()
            return bases

        def pair(p, bases):
            g0 = p * 2

            @pl.when(g0 + 1 < nsup)
            def _():
                load(g0 + 1, 1).start()

            load(g0, 0).wait()
            bases = process(g0, 0, bases)

            @pl.when(g0 + 2 < nsup)
            def _():
                load(g0 + 2, 0).start()

            load(g0 + 1, 1).wait()
            bases = process(g0 + 1, 1, bases)
            return bases

        lax.fori_loop(0, nsup // 2, pair, base_s)
        pltpu.sync_copy(dest_v, dest_hbm.at[pl.ds(base, per)])

    return scatter_kernel


_LOG2E10 = 14.426950408889634


def _celu_shift(x):
    return jnp.maximum(x, 0.0) + 0.1 * jnp.exp2(jnp.minimum(x, 0.0) * _LOG2E10)


def _mlp_block_kernel(tbl_ref, x_ref, w0_ref, b0_ref, w1_ref, b1_ref,
                      w2_ref, b2_ref, out_ref):
    x = x_ref[...].astype(jnp.bfloat16)
    h = _celu_shift(lax.dot_general(x, w0_ref[0], (((1,), (1,)), ((), ())),
                                    preferred_element_type=jnp.float32)
                    + b0_ref[0])
    h = _celu_shift(lax.dot_general(h.astype(jnp.bfloat16), w1_ref[0],
                                    (((1,), (1,)), ((), ())),
                                    preferred_element_type=jnp.float32)
                    + b1_ref[0])
    h = _celu_shift(lax.dot_general(h.astype(jnp.bfloat16), w2_ref[0],
                                    (((1,), (1,)), ((), ())),
                                    preferred_element_type=jnp.float32)
                    + b2_ref[0])
    out_ref[...] = h - 0.1


def _grouped_mlp(sorted_aevs, tbl, stacked):
    npad, aev_dim = sorted_aevs.shape
    nblk = npad // BLK
    w0, b0, w1, b1, w2, b2 = stacked
    n_out = w2.shape[1]

    grid_spec = pltpu.PrefetchScalarGridSpec(
        num_scalar_prefetch=1,
        grid=(nblk,),
        in_specs=[
            pl.BlockSpec((BLK, aev_dim), lambda i, t: (i, 0)),
            pl.BlockSpec((1,) + w0.shape[1:], lambda i, t: (t[i], 0, 0)),
            pl.BlockSpec((1, 1, b0.shape[2]), lambda i, t: (t[i], 0, 0)),
            pl.BlockSpec((1,) + w1.shape[1:], lambda i, t: (t[i], 0, 0)),
            pl.BlockSpec((1, 1, b1.shape[2]), lambda i, t: (t[i], 0, 0)),
            pl.BlockSpec((1,) + w2.shape[1:], lambda i, t: (t[i], 0, 0)),
            pl.BlockSpec((1, 1, b2.shape[2]), lambda i, t: (t[i], 0, 0)),
        ],
        out_specs=pl.BlockSpec((BLK, n_out), lambda i, t: (i, 0)),
    )
    return pl.pallas_call(
        _mlp_block_kernel,
        grid_spec=grid_spec,
        out_shape=jax.ShapeDtypeStruct((npad, n_out), jnp.float32),
    )(tbl, sorted_aevs, w0, b0, w1, b1, w2, b2)


def _make_gather_kernel(n, n_feats, n_pad, npad):
    per = n // NTILES
    sup = 128
    nsup = per // sup

    @functools.partial(
        pl.kernel,
        out_type=jax.ShapeDtypeStruct((n, n_pad), jnp.float32),
        mesh=_mesh(),
        scratch_types=[
            pltpu.VMEM((per,), jnp.int32),
            pltpu.VMEM((2, sup, n_pad), jnp.float32),
            pltpu.SemaphoreType.DMA,
            pltpu.SemaphoreType.DMA,
            pltpu.SemaphoreType.DMA,
        ],
    )
    def gather_kernel(feats_hbm, dest_hbm, out_hbm, dest_v, rows_v,
                      sem_a, sem_b, sem_o):
        wid = _wid()
        base = wid * per
        pltpu.sync_copy(dest_hbm.at[pl.ds(base, per)], dest_v)

        def gath(g, buf):
            return pltpu.make_async_copy(
                feats_hbm.at[dest_v.at[pl.ds(g * sup, sup)]], rows_v.at[buf],
                sem_a if buf == 0 else sem_b)

        def store(g, buf):
            return pltpu.make_async_copy(
                rows_v.at[buf], out_hbm.at[pl.ds(base + g * sup, sup)], sem_o)

        gath(0, 0).start()

        def pair(p, _):
            g0 = p * 2

            @pl.when(g0 + 1 < nsup)
            def _():
                gath(g0 + 1, 1).start()

            gath(g0, 0).wait()
            store(g0, 0).start()
            store(g0, 0).wait()

            @pl.when(g0 + 2 < nsup)
            def _():
                gath(g0 + 2, 0).start()

            gath(g0 + 1, 1).wait()
            store(g0 + 1, 1).start()
            store(g0 + 1, 1).wait()
            return 0

        lax.fori_loop(0, nsup // 2, pair, 0)

    return gather_kernel


def kernel(species, aevs, W0_s0, b0_s0, W1_s0, b1_s0, W2_s0, b2_s0,
           W0_s1, b0_s1, W1_s1, b1_s1, W2_s1, b2_s1,
           W0_s2, b0_s2, W1_s2, b1_s2, W2_s2, b2_s2,
           W0_s3, b0_s3, W1_s3, b1_s3, W2_s3, b2_s3):
    b, a = species.shape
    n = b * a
    aev_dim = aevs.shape[-1]
    n_feats = W2_s0.shape[0]

    spec_flat = species.reshape(n)
    flat = aevs.reshape(n, aev_dim)
    weights = (W0_s0, b0_s0, W1_s0, b1_s0, W2_s0, b2_s0,
               W0_s1, b0_s1, W1_s1, b1_s1, W2_s1, b2_s1,
               W0_s2, b0_s2, W1_s2, b1_s2, W2_s2, b2_s2,
               W0_s3, b0_s3, W1_s3, b1_s3, W2_s3, b2_s3)

    n_pad = 128
    h1p, h2p = 160, 128
    per_sp = [weights[6 * s:6 * s + 6] for s in range(N_SPECIES)]

    def pad2(w, r, c):
        return jnp.pad(w, ((0, r - w.shape[0]), (0, c - w.shape[1])))

    def pad1(v, r):
        return jnp.pad(v, ((0, r - v.shape[0]),))

    w0 = jnp.stack([pad2(p[0], h1p, aev_dim) for p in per_sp])
    b0 = jnp.stack([pad1(p[1], h1p) for p in per_sp])
    w1 = jnp.stack([pad2(p[2], h2p, h1p) for p in per_sp])
    b1 = jnp.stack([pad1(p[3], h2p) for p in per_sp])
    w2 = jnp.stack([pad2(p[4], n_pad, h2p) for p in per_sp])
    b2 = jnp.stack([pad1(p[5], n_pad) for p in per_sp])
    b1 = b1 - 0.1 * jnp.sum(w1, axis=2)
    b2 = b2 - 0.1 * jnp.sum(w2, axis=2)
    stacked = (w0.astype(jnp.bfloat16), b0[:, None, :],
               w1.astype(jnp.bfloat16), b1[:, None, :],
               w2.astype(jnp.bfloat16), b2[:, None, :])

    npad = n + N_SPECIES * BLK
    tbl_len = ((npad // BLK) + LANES - 1) // LANES * LANES

    counts = _make_counts_kernel(n)(spec_flat)
    sorted_aevs, dest, tbl = _make_scatter_kernel(n, aev_dim, npad, tbl_len)(
        spec_flat, flat, counts)
    sorted_feats = _grouped_mlp(sorted_aevs, tbl, stacked)
    final = _make_gather_kernel(n, n_feats, n_pad, npad)(sorted_feats, dest)

    return species, final[:, :n_feats].reshape(b, a, n_feats)


# speedup vs baseline: 1.3649x; 1.0198x over previous
import functools

import jax
import jax.numpy as jnp
from jax import lax
from jax.experimental import pallas as pl
from jax.experimental.pallas import tpu as pltpu
from jax.experimental.pallas import tpu_sc as plsc

N_SPECIES = 4
LANES = 16
NTILES = 32
BLK = 4096


def _mesh():
    return plsc.VectorSubcoreMesh(core_axis_name="c", subcore_axis_name="s")


def _wid():
    return lax.axis_index("s") * 2 + lax.axis_index("c")


def _lane_iota():
    return lax.iota(jnp.int32, LANES)


def _take(x, idx):
    dnums = lax.GatherDimensionNumbers(
        offset_dims=(), collapsed_slice_dims=(0,), start_index_map=(0,))
    return lax.gather(x, idx[:, None], dnums, slice_sizes=(1,),
                      mode=lax.GatherScatterMode.PROMISE_IN_BOUNDS)


def _incl_prefix(x):
    iot = _lane_iota()
    for sh in (1, 2, 4, 8):
        shifted = _take(x, jnp.maximum(iot - sh, 0))
        x = x + jnp.where(iot >= sh, shifted, 0.0)
    return x


def _lanesum_splat(x):
    return _take(_incl_prefix(x), jnp.full((LANES,), LANES - 1, jnp.int32))


def _make_counts_kernel(n):
    per = n // NTILES

    @functools.partial(
        pl.kernel,
        out_type=jax.ShapeDtypeStruct((NTILES, N_SPECIES, LANES), jnp.float32),
        mesh=_mesh(),
        scratch_types=[
            pltpu.VMEM((per,), jnp.int32),
            pltpu.VMEM((N_SPECIES, LANES), jnp.float32),
        ],
    )
    def counts_kernel(spec_hbm, counts_hbm, spec_v, cnt_v):
        wid = _wid()
        base = wid * per
        pltpu.sync_copy(spec_hbm.at[pl.ds(base, per)], spec_v)

        def body(i, carry):
            v = spec_v[pl.ds(i * LANES, LANES)]
            return tuple(carry[s] + jnp.where(v == s, 1.0, 0.0)
                         for s in range(N_SPECIES))

        zeros = jnp.zeros((LANES,), jnp.float32)
        cnts = lax.fori_loop(0, per // LANES, body, (zeros,) * N_SPECIES)
        for s in range(N_SPECIES):
            cnt_v[s, :] = cnts[s]
        pltpu.sync_copy(cnt_v, counts_hbm.at[wid])

    return counts_kernel


def _make_scatter_kernel(n, aev_dim, npad, tbl_len):
    per = n // NTILES
    sup = 128
    nsup = per // sup
    nchunk = sup // LANES

    @functools.partial(
        pl.kernel,
        out_type=(
            jax.ShapeDtypeStruct((npad, aev_dim), jnp.float32),
            jax.ShapeDtypeStruct((n,), jnp.int32),
            jax.ShapeDtypeStruct((tbl_len,), jnp.int32),
        ),
        mesh=_mesh(),
        scratch_types=[
            pltpu.VMEM((per,), jnp.int32),
            pltpu.VMEM((per,), jnp.int32),
            pltpu.VMEM((NTILES, N_SPECIES, LANES), jnp.float32),
            pltpu.VMEM((2, sup, aev_dim), jnp.float32),
            pltpu.VMEM((tbl_len,), jnp.int32),
            pltpu.SemaphoreType.DMA,
            pltpu.SemaphoreType.DMA,
            pltpu.SemaphoreType.DMA,
        ],
    )
    def scatter_kernel(spec_hbm, aev_hbm, counts_hbm,
                       sorted_hbm, dest_hbm, tbl_hbm,
                       spec_v, dest_v, cnt_v, rows_v, tbl_v,
                       sem_a, sem_b, sem_s):
        wid = _wid()
        base = wid * per
        pltpu.sync_copy(spec_hbm.at[pl.ds(base, per)], spec_v)
        pltpu.sync_copy(counts_hbm, cnt_v)

        zero = jnp.zeros((LANES,), jnp.float32)
        tot = [zero] * N_SPECIES
        before = [zero] * N_SPECIES
        for w in range(NTILES):
            sel = jnp.where(jnp.int32(w) < wid, 1.0, 0.0)
            for s in range(N_SPECIES):
                v = cnt_v[w, s, :]
                tot[s] = tot[s] + v
                before[s] = before[s] + v * sel
        tot_s = [_lanesum_splat(tot[s]) for s in range(N_SPECIES)]
        blk_f = jnp.float32(BLK)
        start_s = []
        acc = jnp.zeros((LANES,), jnp.float32)
        for s in range(N_SPECIES):
            start_s.append(acc)
            r = lax.rem(tot_s[s], blk_f)
            acc = acc + tot_s[s] + jnp.where(r > 0, blk_f - r, 0.0)
        base_s = tuple(start_s[s] + _lanesum_splat(before[s])
                       for s in range(N_SPECIES))

        iot = lax.iota(jnp.int32, LANES)
        for t in range(tbl_len // LANES):
            brow = (iot + t * LANES).astype(jnp.float32) * blk_f
            sp = jnp.zeros((LANES,), jnp.float32)
            for s in range(1, N_SPECIES):
                sp = sp + jnp.where(brow >= start_s[s], 1.0, 0.0)
            tbl_v[pl.ds(t * LANES, LANES)] = sp.astype(jnp.int32)

        @pl.when(wid == 0)
        def _():
            pltpu.sync_copy(tbl_v, tbl_hbm)

        def load(g, buf):
            return pltpu.make_async_copy(
                aev_hbm.at[pl.ds(base + g * sup, sup)], rows_v.at[buf],
                sem_a if buf == 0 else sem_b)

        load(0, 0).start()

        def process(g, buf, bases):
            handles = []
            for j in range(nchunk):
                c = g * nchunk + j
                v = spec_v[pl.ds(c * LANES, LANES)]
                dest = jnp.zeros((LANES,), jnp.float32)
                new_bases = []
                for s in range(N_SPECIES):
                    mi = jnp.where(v == s, 1.0, 0.0)
                    ipfx = _incl_prefix(mi)
                    dest = dest + mi * (bases[s] + ipfx - mi)
                    new_bases.append(
                        bases[s] + _take(ipfx, jnp.full((LANES,), LANES - 1,
                                                        jnp.int32)))
                bases = tuple(new_bases)
                dest_i = dest.astype(jnp.int32)
                dest_v[pl.ds(c * LANES, LANES)] = dest_i
                handles.append(pltpu.async_copy(
                    rows_v.at[buf].at[pl.ds(j * LANES, LANES)],
                    sorted_hbm.at[dest_i], sem_s))
            for h in handles:
                h.wait()
            return bases

        def pair(p, bases):
            g0 = p * 2

            @pl.when(g0 + 1 < nsup)
            def _():
                load(g0 + 1, 1).start()

            load(g0, 0).wait()
            bases = process(g0, 0, bases)

            @pl.when(g0 + 2 < nsup)
            def _():
                load(g0 + 2, 0).start()

            load(g0 + 1, 1).wait()
            bases = process(g0 + 1, 1, bases)
            return bases

        lax.fori_loop(0, nsup // 2, pair, base_s)
        pltpu.sync_copy(dest_v, dest_hbm.at[pl.ds(base, per)])

    return scatter_kernel


_LOG2E10 = 14.426950408889634


def _celu_shift(x):
    return jnp.maximum(x, 0.0) + 0.1 * jnp.exp2(jnp.minimum(x, 0.0) * _LOG2E10)


def _mlp_block_kernel(tbl_ref, x_ref, w0_ref, b0_ref, w1_ref, b1_ref,
                      w2_ref, b2_ref, out_ref):
    x = x_ref[...].astype(jnp.bfloat16)
    h = _celu_shift(lax.dot_general(x, w0_ref[0], (((1,), (1,)), ((), ())),
                                    preferred_element_type=jnp.float32)
                    + b0_ref[0])
    h = _celu_shift(lax.dot_general(h.astype(jnp.bfloat16), w1_ref[0],
                                    (((1,), (1,)), ((), ())),
                                    preferred_element_type=jnp.float32)
                    + b1_ref[0])
    h = _celu_shift(lax.dot_general(h.astype(jnp.bfloat16), w2_ref[0],
                                    (((1,), (1,)), ((), ())),
                                    preferred_element_type=jnp.float32)
                    + b2_ref[0])
    out_ref[...] = h - 0.1


def _grouped_mlp(sorted_aevs, tbl, stacked):
    npad, aev_dim = sorted_aevs.shape
    nblk = npad // BLK
    w0, b0, w1, b1, w2, b2 = stacked
    n_out = w2.shape[1]

    grid_spec = pltpu.PrefetchScalarGridSpec(
        num_scalar_prefetch=1,
        grid=(nblk,),
        in_specs=[
            pl.BlockSpec((BLK, aev_dim), lambda i, t: (i, 0)),
            pl.BlockSpec((1,) + w0.shape[1:], lambda i, t: (t[i], 0, 0)),
            pl.BlockSpec((1, 1, b0.shape[2]), lambda i, t: (t[i], 0, 0)),
            pl.BlockSpec((1,) + w1.shape[1:], lambda i, t: (t[i], 0, 0)),
            pl.BlockSpec((1, 1, b1.shape[2]), lambda i, t: (t[i], 0, 0)),
            pl.BlockSpec((1,) + w2.shape[1:], lambda i, t: (t[i], 0, 0)),
            pl.BlockSpec((1, 1, b2.shape[2]), lambda i, t: (t[i], 0, 0)),
        ],
        out_specs=pl.BlockSpec((BLK, n_out), lambda i, t: (i, 0)),
    )
    return pl.pallas_call(
        _mlp_block_kernel,
        grid_spec=grid_spec,
        out_shape=jax.ShapeDtypeStruct((npad, n_out), jnp.float32),
    )(tbl, sorted_aevs, w0, b0, w1, b1, w2, b2)


def _make_gather_kernel(n, n_feats, n_pad, npad):
    per = n // NTILES
    sup = 128
    nsup = per // sup

    @functools.partial(
        pl.kernel,
        out_type=jax.ShapeDtypeStruct((n, n_pad), jnp.float32),
        mesh=_mesh(),
        scratch_types=[
            pltpu.VMEM((per,), jnp.int32),
            pltpu.VMEM((2, sup, n_pad), jnp.float32),
            pltpu.SemaphoreType.DMA,
            pltpu.SemaphoreType.DMA,
            pltpu.SemaphoreType.DMA,
        ],
    )
    def gather_kernel(feats_hbm, dest_hbm, out_hbm, dest_v, rows_v,
                      sem_a, sem_b, sem_o):
        wid = _wid()
        base = wid * per
        pltpu.sync_copy(dest_hbm.at[pl.ds(base, per)], dest_v)

        def gath(g, buf):
            return pltpu.make_async_copy(
                feats_hbm.at[dest_v.at[pl.ds(g * sup, sup)]], rows_v.at[buf],
                sem_a if buf == 0 else sem_b)

        def store(g, buf):
            return pltpu.make_async_copy(
                rows_v.at[buf], out_hbm.at[pl.ds(base + g * sup, sup)], sem_o)

        gath(0, 0).start()

        def pair(p, _):
            g0 = p * 2

            @pl.when(g0 + 1 < nsup)
            def _():
                gath(g0 + 1, 1).start()

            gath(g0, 0).wait()
            store(g0, 0).start()
            store(g0, 0).wait()

            @pl.when(g0 + 2 < nsup)
            def _():
                gath(g0 + 2, 0).start()

            gath(g0 + 1, 1).wait()
            store(g0 + 1, 1).start()
            store(g0 + 1, 1).wait()
            return 0

        lax.fori_loop(0, nsup // 2, pair, 0)

    return gather_kernel


def kernel(species, aevs, W0_s0, b0_s0, W1_s0, b1_s0, W2_s0, b2_s0,
           W0_s1, b0_s1, W1_s1, b1_s1, W2_s1, b2_s1,
           W0_s2, b0_s2, W1_s2, b1_s2, W2_s2, b2_s2,
           W0_s3, b0_s3, W1_s3, b1_s3, W2_s3, b2_s3):
    b, a = species.shape
    n = b * a
    aev_dim = aevs.shape[-1]
    n_feats = W2_s0.shape[0]

    spec_flat = species.reshape(n)
    flat = aevs.reshape(n, aev_dim)
    weights = (W0_s0, b0_s0, W1_s0, b1_s0, W2_s0, b2_s0,
               W0_s1, b0_s1, W1_s1, b1_s1, W2_s1, b2_s1,
               W0_s2, b0_s2, W1_s2, b1_s2, W2_s2, b2_s2,
               W0_s3, b0_s3, W1_s3, b1_s3, W2_s3, b2_s3)

    n_pad = 128
    h1p, h2p = 160, 128
    per_sp = [weights[6 * s:6 * s + 6] for s in range(N_SPECIES)]

    def pad2(w, r, c):
        return jnp.pad(w, ((0, r - w.shape[0]), (0, c - w.shape[1])))

    def pad1(v, r):
        return jnp.pad(v, ((0, r - v.shape[0]),))

    w0 = jnp.stack([pad2(p[0], h1p, aev_dim) for p in per_sp])
    b0 = jnp.stack([pad1(p[1], h1p) for p in per_sp])
    w1 = jnp.stack([pad2(p[2], h2p, h1p) for p in per_sp])
    b1 = jnp.stack([pad1(p[3], h2p) for p in per_sp])
    w2 = jnp.stack([pad2(p[4], n_pad, h2p) for p in per_sp])
    b2 = jnp.stack([pad1(p[5], n_pad) for p in per_sp])
    b1 = b1 - 0.1 * jnp.sum(w1, axis=2)
    b2 = b2 - 0.1 * jnp.sum(w2, axis=2)
    stacked = (w0.astype(jnp.bfloat16), b0[:, None, :],
               w1.astype(jnp.bfloat16), b1[:, None, :],
               w2.astype(jnp.bfloat16), b2[:, None, :])

    npad = n + N_SPECIES * BLK
    tbl_len = ((npad // BLK) + LANES - 1) // LANES * LANES

    counts = _make_counts_kernel(n)(spec_flat)
    sorted_aevs, dest, tbl = _make_scatter_kernel(n, aev_dim, npad, tbl_len)(
        spec_flat, flat, counts)
    sorted_feats = _grouped_mlp(sorted_aevs, tbl, stacked)
    final = _make_gather_kernel(n, n_feats, n_pad, npad)(sorted_feats, dest)

    return species, final[:, :n_feats].reshape(b, a, n_feats)
